# Initial kernel scaffold; baseline (speedup 1.0000x reference)
import jax
import jax.numpy as jnp
from jax.experimental import pallas as pl


def kernel(graph_input, graph_label, W1, b1, W2, b2):
    raise NotImplementedError("write your pallas kernel here")


# speedup vs baseline: 9.5924x; 9.5924x over previous
import functools

import jax
import jax.numpy as jnp
from jax import lax
from jax.experimental import pallas as pl
from jax.experimental.pallas import tpu as pltpu
from jax.experimental.pallas import tpu_sc as plsc

G = 4
N = 4096
Z = 128
C = 8
CG = G * C
NC = 2
NS = 16
NW = NC * NS
ROWS = G * N
RPW = ROWS // NW
CH = 128
NCHUNK = RPW // CH
LANES = 16
FB = Z // LANES

_mesh = plsc.VectorSubcoreMesh(core_axis_name="c", subcore_axis_name="s")


def _seg_sum_body(x_hbm, idx_hbm, out_hbm, xbuf, idxbuf, acc, ident, acc_sh):
    c = lax.axis_index("c")
    s = lax.axis_index("s")
    w = s * NC + c
    base = w * RPW

    zero = jnp.zeros((LANES,), jnp.float32)
    for r in range(CG):
        for f in range(FB):
            acc[r, pl.ds(f * LANES, LANES)] = zero
    iota = lax.iota(jnp.int32, LANES)
    ident[pl.ds(0, LANES)] = iota
    ident[pl.ds(LANES, LANES)] = iota + LANES

    @pl.when(s == 0)
    def _():
        pltpu.sync_copy(acc, acc_sh)

    plsc.subcore_barrier()

    pltpu.sync_copy(idx_hbm.at[pl.ds(w * NCHUNK, NCHUNK)], idxbuf)
    for j in range(NCHUNK):
        pltpu.sync_copy(x_hbm.at[pl.ds(base + j * CH, CH)], xbuf)

        def body(b, carry, j=j):
            cv = idxbuf[j, pl.ds(b * LANES, LANES)]
            for t in range(LANES):
                cls = cv[t]
                i = b * LANES + t
                for f in range(FB):
                    v = xbuf[i, pl.ds(f * LANES, LANES)]
                    plsc.addupdate(acc.at[cls, pl.ds(f * LANES, LANES)], v)
            return carry

        lax.fori_loop(0, CH // LANES, body, 0)

    pltpu.sync_copy(acc, acc_sh.at[ident], add=True)
    plsc.subcore_barrier()

    @pl.when(s == 0)
    def _():
        pltpu.sync_copy(acc_sh, out_hbm.at[c])


_seg_sum = pl.kernel(
    _seg_sum_body,
    out_type=jax.ShapeDtypeStruct((NC, CG, Z), jnp.float32),
    mesh=_mesh,
    scratch_types=[
        pltpu.VMEM((CH, Z), jnp.float32),
        pltpu.VMEM((NCHUNK, CH), jnp.int32),
        pltpu.VMEM((CG, Z), jnp.float32),
        pltpu.VMEM((CG,), jnp.int32),
        pltpu.VMEM_SHARED((CG, Z), jnp.float32),
    ],
)


def _dense_body(sums_ref, lab_ref, w1_ref, b1_ref, w2_ref, b2_ref, out_ref):
    sums = sums_ref[0] + sums_ref[1]
    lab = lab_ref[...]
    rowc = lax.broadcasted_iota(jnp.int32, (CG, 1), 0) % C
    cnt = jnp.zeros((CG, 1), jnp.float32)
    for cc in range(C):
        n_cc = jnp.sum(jnp.where(lab == cc, 1.0, 0.0))
        cnt = jnp.where(rowc == cc, n_cc, cnt)
    m = sums / jnp.maximum(cnt, 1.0)
    h1 = jnp.dot(m, w1_ref[...], preferred_element_type=jnp.float32)
    h1 = jnp.maximum(h1 + b1_ref[...], 0.0)
    h2 = jnp.dot(h1, w2_ref[...], preferred_element_type=jnp.float32)
    out_ref[...] = h2 + b2_ref[...]


_dense = pl.pallas_call(
    _dense_body,
    out_shape=jax.ShapeDtypeStruct((CG, Z), jnp.float32),
)


def _gather_add_body(x_hbm, idx_hbm, h2_hbm, out_hbm, xbuf, gbuf, idxbuf, sem):
    c = lax.axis_index("c")
    s = lax.axis_index("s")
    w = s * NC + c
    base = w * RPW

    pltpu.sync_copy(idx_hbm.at[pl.ds(w * NCHUNK, NCHUNK)], idxbuf)
    for j in range(NCHUNK):
        pltpu.sync_copy(x_hbm.at[pl.ds(base + j * CH, CH)], xbuf)
        pltpu.async_copy(h2_hbm.at[idxbuf.at[j]], gbuf, sem).wait()

        def body(i, carry):
            for f in range(FB):
                v = gbuf[i, pl.ds(f * LANES, LANES)]
                plsc.addupdate(xbuf.at[i, pl.ds(f * LANES, LANES)], v)
            return carry

        lax.fori_loop(0, CH, body, 0)
        pltpu.sync_copy(xbuf, out_hbm.at[pl.ds(base + j * CH, CH)])


_gather_add = pl.kernel(
    _gather_add_body,
    out_type=jax.ShapeDtypeStruct((ROWS, Z), jnp.float32),
    mesh=_mesh,
    scratch_types=[
        pltpu.VMEM((CH, Z), jnp.float32),
        pltpu.VMEM((CH, Z), jnp.float32),
        pltpu.VMEM((NCHUNK, CH), jnp.int32),
        pltpu.SemaphoreType.DMA,
    ],
)


def kernel(graph_input, graph_label, W1, b1, W2, b2):
    x = graph_input.reshape(ROWS, Z)
    idx = (graph_label[None, :].astype(jnp.int32)
           + C * jnp.arange(G, dtype=jnp.int32)[:, None]).reshape(ROWS // CH, CH)
    sums2 = _seg_sum(x, idx)
    lab2d = graph_label.astype(jnp.int32).reshape(CG, CH)
    h2 = _dense(sums2, lab2d, W1, b1.reshape(1, 4 * Z), W2, b2.reshape(1, Z))
    out = _gather_add(x, idx, h2)
    return out.reshape(G, N, Z)


# speedup vs baseline: 12.4284x; 1.2956x over previous
import functools

import jax
import jax.numpy as jnp
from jax import lax
from jax.experimental import pallas as pl
from jax.experimental.pallas import tpu as pltpu
from jax.experimental.pallas import tpu_sc as plsc

G = 4
N = 4096
Z = 128
C = 8
CG = G * C
NC = 2
NS = 16
NW = NC * NS
ROWS = G * N
RPW = ROWS // NW
CH = 256
NCHUNK = RPW // CH
LANES = 16
FB = Z // LANES
IDXR = 128

_mesh = plsc.VectorSubcoreMesh(core_axis_name="c", subcore_axis_name="s")


def _accum_chunk(xbuf, idxbuf, acc, chunk):
    def body(b, carry):
        jrow = b // (IDXR // LANES)
        goff = b % (IDXR // LANES)
        cv = idxbuf[jrow, pl.ds(goff * LANES, LANES)]
        for t in range(LANES):
            cls = cv[t]
            i = (b % (CH // LANES)) * LANES + t
            for f in range(FB):
                v = xbuf[i, pl.ds(f * LANES, LANES)]
                plsc.addupdate(acc.at[cls, pl.ds(f * LANES, LANES)], v)
        return carry

    lo = chunk * (CH // LANES)
    lax.fori_loop(lo, lo + CH // LANES, body, 0)


def _seg_sum_body(x_hbm, idx_hbm, out_hbm, xbuf0, xbuf1, idxbuf, acc, ident,
                  acc_sh, sem0, sem1):
    c = lax.axis_index("c")
    s = lax.axis_index("s")
    w = s * NC + c
    base = w * RPW

    cp0 = pltpu.async_copy(x_hbm.at[pl.ds(base, CH)], xbuf0, sem0)
    cp1 = pltpu.async_copy(x_hbm.at[pl.ds(base + CH, CH)], xbuf1, sem1)
    pltpu.sync_copy(idx_hbm.at[pl.ds(w * (RPW // IDXR), RPW // IDXR)], idxbuf)

    zero = jnp.zeros((LANES,), jnp.float32)
    for r in range(CG):
        for f in range(FB):
            acc[r, pl.ds(f * LANES, LANES)] = zero
    iota = lax.iota(jnp.int32, LANES)
    ident[pl.ds(0, LANES)] = iota
    ident[pl.ds(LANES, LANES)] = iota + LANES

    @pl.when(s == 0)
    def _():
        pltpu.sync_copy(acc, acc_sh)

    plsc.subcore_barrier()

    cp0.wait()
    _accum_chunk(xbuf0, idxbuf, acc, 0)
    cp1.wait()
    _accum_chunk(xbuf1, idxbuf, acc, 1)

    pltpu.sync_copy(acc, acc_sh.at[ident], add=True)
    plsc.subcore_barrier()

    @pl.when(s == 0)
    def _():
        pltpu.sync_copy(acc_sh, out_hbm.at[c])


_seg_sum = pl.kernel(
    _seg_sum_body,
    out_type=jax.ShapeDtypeStruct((NC, CG, Z), jnp.float32),
    mesh=_mesh,
    scratch_types=[
        pltpu.VMEM((CH, Z), jnp.float32),
        pltpu.VMEM((CH, Z), jnp.float32),
        pltpu.VMEM((RPW // IDXR, IDXR), jnp.int32),
        pltpu.VMEM((CG, Z), jnp.float32),
        pltpu.VMEM((CG,), jnp.int32),
        pltpu.VMEM_SHARED((CG, Z), jnp.float32),
        pltpu.SemaphoreType.DMA,
        pltpu.SemaphoreType.DMA,
    ],
)


def _dense_body(sums_ref, lab_ref, w1_ref, b1_ref, w2_ref, b2_ref, out_ref):
    sums = sums_ref[0] + sums_ref[1]
    lab = lab_ref[...]
    rowc = lax.broadcasted_iota(jnp.int32, (CG, 1), 0) % C
    cnt = jnp.zeros((CG, 1), jnp.float32)
    for cc in range(C):
        n_cc = jnp.sum(jnp.where(lab == cc, 1.0, 0.0))
        cnt = jnp.where(rowc == cc, n_cc, cnt)
    m = sums / jnp.maximum(cnt, 1.0)
    h1 = jnp.dot(m, w1_ref[...], preferred_element_type=jnp.float32)
    h1 = jnp.maximum(h1 + b1_ref[...], 0.0)
    h2 = jnp.dot(h1, w2_ref[...], preferred_element_type=jnp.float32)
    out_ref[...] = h2 + b2_ref[...]


_dense = pl.pallas_call(
    _dense_body,
    out_shape=jax.ShapeDtypeStruct((CG, Z), jnp.float32),
)


def _gadd_chunk(xbuf, idxbuf, h2buf, chunk):
    def body(b, carry):
        jrow = b // (IDXR // LANES)
        goff = b % (IDXR // LANES)
        cv = idxbuf[jrow, pl.ds(goff * LANES, LANES)]
        for t in range(LANES):
            cls = cv[t]
            i = (b % (CH // LANES)) * LANES + t
            for f in range(FB):
                v = h2buf[cls, pl.ds(f * LANES, LANES)]
                plsc.addupdate(xbuf.at[i, pl.ds(f * LANES, LANES)], v)
        return carry

    lo = chunk * (CH // LANES)
    lax.fori_loop(lo, lo + CH // LANES, body, 0)


def _gather_add_body(x_hbm, idx_hbm, h2_hbm, out_hbm, xbuf0, xbuf1, idxbuf,
                     h2buf, sem0, sem1, osem):
    c = lax.axis_index("c")
    s = lax.axis_index("s")
    w = s * NC + c
    base = w * RPW

    cp0 = pltpu.async_copy(x_hbm.at[pl.ds(base, CH)], xbuf0, sem0)
    cp1 = pltpu.async_copy(x_hbm.at[pl.ds(base + CH, CH)], xbuf1, sem1)
    pltpu.sync_copy(h2_hbm, h2buf)
    pltpu.sync_copy(idx_hbm.at[pl.ds(w * (RPW // IDXR), RPW // IDXR)], idxbuf)

    cp0.wait()
    _gadd_chunk(xbuf0, idxbuf, h2buf, 0)
    st0 = pltpu.async_copy(xbuf0, out_hbm.at[pl.ds(base, CH)], osem)
    cp1.wait()
    _gadd_chunk(xbuf1, idxbuf, h2buf, 1)
    st1 = pltpu.async_copy(xbuf1, out_hbm.at[pl.ds(base + CH, CH)], osem)
    st0.wait()
    st1.wait()


_gather_add = pl.kernel(
    _gather_add_body,
    out_type=jax.ShapeDtypeStruct((ROWS, Z), jnp.float32),
    mesh=_mesh,
    scratch_types=[
        pltpu.VMEM((CH, Z), jnp.float32),
        pltpu.VMEM((CH, Z), jnp.float32),
        pltpu.VMEM((RPW // IDXR, IDXR), jnp.int32),
        pltpu.VMEM((CG, Z), jnp.float32),
        pltpu.SemaphoreType.DMA,
        pltpu.SemaphoreType.DMA,
        pltpu.SemaphoreType.DMA,
    ],
)


def kernel(graph_input, graph_label, W1, b1, W2, b2):
    x = graph_input.reshape(ROWS, Z)
    idx = (graph_label[None, :].astype(jnp.int32)
           + C * jnp.arange(G, dtype=jnp.int32)[:, None]).reshape(ROWS // IDXR, IDXR)
    sums2 = _seg_sum(x, idx)
    lab2d = graph_label.astype(jnp.int32).reshape(CG, IDXR)
    h2 = _dense(sums2, lab2d, W1, b1.reshape(1, 4 * Z), W2, b2.reshape(1, Z))
    out = _gather_add(x, idx, h2)
    return out.reshape(G, N, Z)
